# baseline (speedup 1.0000x reference)
import functools

import jax
import jax.numpy as jnp
from jax.experimental import pallas as pl
from jax.experimental.pallas import tpu as pltpu

N_CLS = 600
N_CTX = 5
D = 512
SEQ = 77
SUF = SEQ - 1 - N_CTX
B = 1024


def _body(target_ref, bias_ref, ctx_ref, prefix_ref, suffix_ref, out_ref):
    out_ref[0, 0, :] = prefix_ref[0, 0, :]
    out_ref[0, 1 : 1 + N_CTX, :] = ctx_ref[:, :] + bias_ref[0]
    out_ref[0, 1 + N_CTX :, :] = suffix_ref[0]


@jax.jit
def kernel(bias, target, ctx, token_prefix, token_suffix):
    target = target.astype(jnp.int32)
    grid_spec = pltpu.PrefetchScalarGridSpec(
        num_scalar_prefetch=1,
        grid=(B,),
        in_specs=[
            pl.BlockSpec((1, 1, D), lambda b, tgt: (b, 0, 0)),
            pl.BlockSpec((N_CTX, D), lambda b, tgt: (0, 0)),
            pl.BlockSpec((1, 1, D), lambda b, tgt: (tgt[b], 0, 0)),
            pl.BlockSpec((1, SUF, D), lambda b, tgt: (tgt[b], 0, 0)),
        ],
        out_specs=pl.BlockSpec((1, SEQ, D), lambda b, tgt: (b, 0, 0)),
    )
    fn = pl.pallas_call(
        _body,
        grid_spec=grid_spec,
        out_shape=jax.ShapeDtypeStruct((B, SEQ, D), jnp.float32),
        compiler_params=pltpu.CompilerParams(
            dimension_semantics=("arbitrary",)
        ),
    )
    return fn(target, bias[:, None, :], ctx, token_prefix, token_suffix)


# speedup vs baseline: 4.6593x; 4.6593x over previous
import functools

import jax
import jax.numpy as jnp
from jax import lax
from jax.experimental import pallas as pl
from jax.experimental.pallas import tpu as pltpu
from jax.experimental.pallas import tpu_sc as plsc

N_CLS = 600
N_CTX = 5
D = 512
SEQ = 77
SUF = SEQ - 1 - N_CTX
B = 1024

NC = 2
NS = 16
NW = NC * NS
BPW = B // NW
LANES = 16
CHUNKS = D // LANES

NBUF = 4
LAG = 3

_mesh = plsc.VectorSubcoreMesh(
    core_axis_name="c", subcore_axis_name="s", num_cores=NC, num_subcores=NS
)


@functools.partial(
    pl.kernel,
    out_type=jax.ShapeDtypeStruct((SEQ * B, D), jnp.float32),
    mesh=_mesh,
    scratch_types=[
        pltpu.VMEM((BPW,), jnp.int32),
        pltpu.VMEM((BPW, D), jnp.float32),
        pltpu.VMEM((N_CTX, D), jnp.float32),
        pltpu.VMEM((BPW, D), jnp.float32),
        pltpu.VMEM((BPW, D), jnp.float32),
        pltpu.VMEM((BPW, D), jnp.float32),
        pltpu.VMEM((BPW, D), jnp.float32),
        pltpu.VMEM((BPW, D), jnp.float32),
        pltpu.VMEM((BPW, D), jnp.float32),
        pltpu.SemaphoreType.DMA,
        pltpu.SemaphoreType.DMA,
    ],
)
def _prompt_kernel(
    bias_hbm, target_hbm, ctx_hbm, prefix_hbm, suffix_hbm, out_hbm,
    idx_v, bias_v, ctx_v, h0_v, h1_v, g0_v, g1_v, g2_v, g3_v, gsem, osem,
):
    hbufs = (h0_v, h1_v)
    gbufs = (g0_v, g1_v, g2_v, g3_v)
    wid = lax.axis_index("s") * NC + lax.axis_index("c")
    base = wid * BPW

    pltpu.sync_copy(target_hbm.at[pl.ds(base, BPW)], idx_v)
    pltpu.sync_copy(bias_hbm.at[pl.ds(base, BPW)], bias_v)
    pltpu.sync_copy(ctx_hbm, ctx_v)

    def fire_out(plane, buf):
        pltpu.async_copy(buf, out_hbm.at[pl.ds(plane * B + base, BPW)], osem)

    def drain_out():
        pltpu.make_async_copy(
            gbufs[0], out_hbm.at[pl.ds(base, BPW)], osem
        ).wait()

    def drain_gather():
        pltpu.make_async_copy(
            gbufs[0], out_hbm.at[pl.ds(base, BPW)], gsem
        ).wait()

    pltpu.async_copy(prefix_hbm.at[idx_v], h0_v, gsem).wait()
    fire_out(0, h0_v)

    def compute_ctx_plane(j, hbuf):
        def row(i, carry):
            def chunk(c, carry2):
                o = c * LANES
                hbuf[i, pl.ds(o, LANES)] = (
                    bias_v[i, pl.ds(o, LANES)] + ctx_v[j, pl.ds(o, LANES)]
                )
                return carry2

            lax.fori_loop(0, CHUNKS, chunk, 0, unroll=4)
            return carry

        lax.fori_loop(0, BPW, row, 0)

    for j in range(N_CTX):
        hbuf = hbufs[(j + 1) % 2]
        if j >= 1:
            drain_out()
        compute_ctx_plane(j, hbuf)
        fire_out(1 + j, hbuf)
    drain_out()
    drain_out()

    for step in range(SUF + LAG):
        r = step
        if r < SUF:
            if r >= NBUF:
                drain_out()
            pltpu.async_copy(
                suffix_hbm.at[r].at[idx_v], gbufs[r % NBUF], gsem
            )
        if step >= LAG:
            rr = step - LAG
            drain_gather()
            fire_out(1 + N_CTX + rr, gbufs[rr % NBUF])

    for _ in range(NBUF):
        drain_out()


def kernel(bias, target, ctx, token_prefix, token_suffix):
    target = target.astype(jnp.int32)
    prefix2 = token_prefix.reshape(N_CLS, D)
    suffix_t = jnp.transpose(token_suffix, (1, 0, 2))
    out2 = _prompt_kernel(bias, target, ctx, prefix2, suffix_t)
    return jnp.transpose(out2.reshape(SEQ, B, D), (1, 0, 2))


# speedup vs baseline: 5.2969x; 1.1368x over previous
import functools

import jax
import jax.numpy as jnp
from jax import lax
from jax.experimental import pallas as pl
from jax.experimental.pallas import tpu as pltpu
from jax.experimental.pallas import tpu_sc as plsc

N_CLS = 600
N_CTX = 5
D = 512
SEQ = 77
SUF = SEQ - 1 - N_CTX
B = 1024

NC = 2
NS = 16
NW = NC * NS
BPW = B // NW
LANES = 16
CHUNKS = D // LANES

NBUF = 4
LAG = 3

_mesh = plsc.VectorSubcoreMesh(
    core_axis_name="c", subcore_axis_name="s", num_cores=NC, num_subcores=NS
)


@functools.partial(
    pl.kernel,
    out_type=jax.ShapeDtypeStruct((SEQ * B, D), jnp.float32),
    mesh=_mesh,
    scratch_types=[
        pltpu.VMEM((BPW,), jnp.int32),
        pltpu.VMEM((BPW, D), jnp.float32),
        pltpu.VMEM((N_CTX, D), jnp.float32),
        pltpu.VMEM((BPW, D), jnp.float32),
        pltpu.VMEM((BPW, D), jnp.float32),
        pltpu.VMEM((BPW, D), jnp.float32),
        pltpu.VMEM((BPW, D), jnp.float32),
        pltpu.VMEM((BPW, D), jnp.float32),
        pltpu.VMEM((BPW, D), jnp.float32),
        pltpu.SemaphoreType.DMA,
        pltpu.SemaphoreType.DMA,
        pltpu.SemaphoreType.DMA,
        pltpu.SemaphoreType.DMA,
    ],
)
def _prompt_kernel(
    bias_hbm, target_hbm, ctx_hbm, prefix_hbm, suffix_hbm, out_hbm,
    idx_v, bias_v, ctx_v, h0_v, h1_v, g0_v, g1_v, g2_v, g3_v,
    gsem, osem, psem, hsem,
):
    hbufs = (h0_v, h1_v)
    gbufs = (g0_v, g1_v, g2_v, g3_v)
    wid = lax.axis_index("s") * NC + lax.axis_index("c")
    base = wid * BPW

    pltpu.sync_copy(target_hbm.at[pl.ds(base, BPW)], idx_v)
    pltpu.sync_copy(bias_hbm.at[pl.ds(base, BPW)], bias_v)
    pltpu.sync_copy(ctx_hbm, ctx_v)

    def fire_out(plane, buf, sem):
        pltpu.async_copy(buf, out_hbm.at[pl.ds(plane * B + base, BPW)], sem)

    def drain(sem):
        pltpu.make_async_copy(
            gbufs[0], out_hbm.at[pl.ds(base, BPW)], sem
        ).wait()

    for r in range(NBUF):
        pltpu.async_copy(suffix_hbm.at[r].at[idx_v], gbufs[r], gsem)
    pcopy = pltpu.async_copy(prefix_hbm.at[idx_v], h0_v, psem)

    def compute_ctx_plane(j, hbuf):
        def chunk(c, carry):
            o = c * LANES
            cc = ctx_v[j, pl.ds(o, LANES)]

            def row(i, carry2):
                hbuf[i, pl.ds(o, LANES)] = bias_v[i, pl.ds(o, LANES)] + cc
                return carry2

            lax.fori_loop(0, BPW, row, 0, unroll=4)
            return carry

        lax.fori_loop(0, CHUNKS, chunk, 0)

    compute_ctx_plane(0, h1_v)
    fire_out(1, h1_v, hsem)
    pcopy.wait()
    fire_out(0, h0_v, hsem)
    for j in range(1, N_CTX):
        hbuf = hbufs[j % 2]
        drain(hsem)
        compute_ctx_plane(j, hbuf)
        fire_out(1 + j, hbuf, hsem)

    for step in range(SUF + LAG):
        r = step
        if NBUF <= r < SUF:
            drain(osem)
            pltpu.async_copy(
                suffix_hbm.at[r].at[idx_v], gbufs[r % NBUF], gsem
            )
        if step >= LAG:
            rr = step - LAG
            drain(gsem)
            fire_out(1 + N_CTX + rr, gbufs[rr % NBUF], osem)

    for _ in range(NBUF):
        drain(osem)
    drain(hsem)
    drain(hsem)


def kernel(bias, target, ctx, token_prefix, token_suffix):
    target = target.astype(jnp.int32)
    prefix2 = token_prefix.reshape(N_CLS, D)
    suffix_t = jnp.transpose(token_suffix, (1, 0, 2))
    out2 = _prompt_kernel(bias, target, ctx, prefix2, suffix_t)
    return jnp.transpose(out2.reshape(SEQ, B, D), (1, 0, 2))


# speedup vs baseline: 5.3391x; 1.0080x over previous
import functools

import jax
import jax.numpy as jnp
from jax import lax
from jax.experimental import pallas as pl
from jax.experimental.pallas import tpu as pltpu
from jax.experimental.pallas import tpu_sc as plsc

N_CLS = 600
N_CTX = 5
D = 512
SEQ = 77
SUF = SEQ - 1 - N_CTX
B = 1024

NC = 2
NS = 16
NW = NC * NS
BPW = B // NW
LANES = 16
CHUNKS = D // LANES

NBUF = 4
LAG = 3

_mesh = plsc.VectorSubcoreMesh(
    core_axis_name="c", subcore_axis_name="s", num_cores=NC, num_subcores=NS
)


@functools.partial(
    pl.kernel,
    out_type=jax.ShapeDtypeStruct((SEQ * B, D), jnp.float32),
    mesh=_mesh,
    scratch_types=[
        pltpu.VMEM((BPW,), jnp.int32),
        pltpu.VMEM((BPW, D), jnp.float32),
        pltpu.VMEM((N_CTX, D), jnp.float32),
        pltpu.VMEM((BPW, D), jnp.float32),
        pltpu.VMEM((BPW, D), jnp.float32),
        pltpu.VMEM((BPW, D), jnp.float32),
        pltpu.VMEM((BPW, D), jnp.float32),
        pltpu.VMEM((BPW, D), jnp.float32),
        pltpu.VMEM((BPW, D), jnp.float32),
        pltpu.SemaphoreType.DMA,
        pltpu.SemaphoreType.DMA,
        pltpu.SemaphoreType.DMA,
        pltpu.SemaphoreType.DMA,
    ],
)
def _prompt_kernel(
    bias_hbm, target_hbm, ctx_hbm, prefix_hbm, suffix_hbm, out_hbm,
    idx_v, bias_v, ctx_v, h0_v, h1_v, g0_v, g1_v, g2_v, g3_v,
    gsem, osem, psem, hsem,
):
    hbufs = (h0_v, h1_v)
    gbufs = (g0_v, g1_v, g2_v, g3_v)
    wid = lax.axis_index("s") * NC + lax.axis_index("c")
    base = wid * BPW

    pltpu.sync_copy(target_hbm.at[pl.ds(base, BPW)], idx_v)
    pltpu.sync_copy(bias_hbm.at[pl.ds(base, BPW)], bias_v)
    pltpu.sync_copy(ctx_hbm, ctx_v)

    def fire_out(plane, buf, sem):
        pltpu.async_copy(buf, out_hbm.at[pl.ds(plane * B + base, BPW)], sem)

    def drain(sem):
        pltpu.make_async_copy(
            gbufs[0], out_hbm.at[pl.ds(base, BPW)], sem
        ).wait()

    for r in range(NBUF):
        pltpu.async_copy(suffix_hbm.at[r].at[idx_v], gbufs[r], gsem)
    pcopy = pltpu.async_copy(prefix_hbm.at[idx_v], h0_v, psem)

    def compute_ctx_plane(j, hbuf):
        def chunk(c, carry):
            o = c * LANES
            cc = ctx_v[j, pl.ds(o, LANES)]

            def row(i, carry2):
                hbuf[i, pl.ds(o, LANES)] = bias_v[i, pl.ds(o, LANES)] + cc
                return carry2

            lax.fori_loop(0, BPW, row, 0, unroll=4)
            return carry

        lax.fori_loop(0, CHUNKS, chunk, 0)

    for step in range(SUF + LAG):
        r = step
        if NBUF <= r < SUF:
            drain(osem)
            pltpu.async_copy(
                suffix_hbm.at[r].at[idx_v], gbufs[r % NBUF], gsem
            )
        if step == 0:
            compute_ctx_plane(0, h1_v)
            fire_out(1, h1_v, hsem)
        elif step == 1:
            pcopy.wait()
            fire_out(0, h0_v, hsem)
        elif 2 <= step <= N_CTX:
            j = step - 1
            hbuf = hbufs[j % 2]
            drain(hsem)
            compute_ctx_plane(j, hbuf)
            fire_out(1 + j, hbuf, hsem)
        if step >= LAG:
            rr = step - LAG
            drain(gsem)
            fire_out(1 + N_CTX + rr, gbufs[rr % NBUF], osem)

    for _ in range(NBUF):
        drain(osem)
    drain(hsem)
    drain(hsem)


def kernel(bias, target, ctx, token_prefix, token_suffix):
    target = target.astype(jnp.int32)
    prefix2 = token_prefix.reshape(N_CLS, D)
    suffix_t = jnp.transpose(token_suffix, (1, 0, 2))
    out2 = _prompt_kernel(bias, target, ctx, prefix2, suffix_t)
    return jnp.transpose(out2.reshape(SEQ, B, D), (1, 0, 2))
